# baseline (speedup 1.0000x reference)
import functools

import jax
import jax.numpy as jnp
from jax import lax
from jax.experimental import pallas as pl
from jax.experimental.pallas import tpu as pltpu
from jax.experimental.pallas import tpu_sc as plsc

BATCH = 16384
FIELDS = 26
EMBED_DIM = 64
B_TOTAL = BATCH * FIELDS
NC, NS = 2, 16
NW = NC * NS
B_PER_W = B_TOTAL // NW
BATCH_PER_W = BATCH // NW
NBUF = 2
CHUNK_B = 8
CHUNK = CHUNK_B * FIELDS
N_CHUNKS = BATCH_PER_W // CHUNK_B
LANES = 16

_MESH = plsc.VectorSubcoreMesh(core_axis_name="c", subcore_axis_name="s")


@functools.partial(
    pl.kernel,
    mesh=_MESH,
    out_type=jax.ShapeDtypeStruct((BATCH, FIELDS, EMBED_DIM), jnp.float32),
    scratch_types=[
        pltpu.VMEM((B_PER_W,), jnp.int32),
        pltpu.VMEM((NBUF, CHUNK_B, FIELDS, EMBED_DIM), jnp.float32),
    ]
    + [pltpu.SemaphoreType.DMA] * (2 * NBUF),
)
def _embed_gather(idx_hbm, table_hbm, out_hbm, idx_v, rows_v, *sems):
    gsems, osems = sems[:NBUF], sems[NBUF:]
    wid = lax.axis_index("s") * NC + lax.axis_index("c")
    base = wid * B_PER_W
    batch_base = wid * BATCH_PER_W

    pltpu.sync_copy(idx_hbm.at[pl.ds(base, B_PER_W)], idx_v)

    def gather_start(i, b):
        def group(g, carry):
            vec = idx_v[pl.ds(i * CHUNK + g * LANES, LANES)]
            for l in range(LANES):
                r = vec[l]
                k = g * LANES + l
                kb = k // FIELDS
                kf = k - kb * FIELDS
                pltpu.make_async_copy(
                    table_hbm.at[r], rows_v.at[b].at[kb].at[kf],
                    gsems[b]).start()
            return carry
        lax.fori_loop(0, CHUNK // LANES, group, 0)

    def gather_wait(b):
        pltpu.make_async_copy(
            out_hbm.at[pl.ds(0, CHUNK_B)], rows_v.at[b], gsems[b]).wait()

    def out_copy(i, b):
        b0 = batch_base + i * CHUNK_B
        return pltpu.make_async_copy(
            rows_v.at[b], out_hbm.at[pl.ds(b0, CHUNK_B)], osems[b])

    def out_start(i, b):
        out_copy(i, b).start()

    def out_wait(i, b):
        out_copy(i, b).wait()

    for b in range(NBUF):
        gather_start(b, b)

    def outer(j, carry):
        for b in range(NBUF):
            i = j * NBUF + b
            bp = (b + NBUF - 1) % NBUF

            @pl.when(jnp.logical_and(i >= 1, i <= N_CHUNKS - NBUF))
            def _():
                out_wait(i - 1, bp)
                gather_start(i - 1 + NBUF, bp)

            gather_wait(b)
            out_start(i, b)
        return carry

    lax.fori_loop(0, N_CHUNKS // NBUF, outer, 0)

    for b in range(NBUF):
        out_wait(N_CHUNKS - NBUF + b, b)


def kernel(embed_input, weight):
    idx = embed_input.reshape(-1).astype(jnp.int32)
    return _embed_gather(idx, weight)
